# baseline (speedup 1.0000x reference)
import functools

import jax
import jax.numpy as jnp
from jax import lax
from jax.experimental import pallas as pl
from jax.experimental.pallas import tpu as pltpu
from jax.experimental.pallas import tpu_sc as plsc

N = 10000
D = 128
H = 32
G = 64
LC = 2
E = 320000

NC = 2
NS = 16
NW = NC * NS

CHUNK = 512
NCHUNK = 20
EPW = NCHUNK * CHUNK
EPAD = EPW * NW
NPAD = 10240
RPW = NPAD // NS

RB = 2000
NBLK = N // RB

_F32 = jnp.float32
_HI = lax.Precision.HIGHEST


def _dense_first_body(x_ref, w1_ref, w2_ref, b_ref, msg_ref, z_ref):
    xb = x_ref[...]
    msg_ref[...] = jnp.dot(xb, w1_ref[...], preferred_element_type=_F32,
                           precision=_HI)
    z_ref[...] = jnp.dot(xb, w2_ref[...], preferred_element_type=_F32,
                         precision=_HI) + b_ref[...]


def _dense_mid_body(p_ref, zp_ref, w1_ref, w2_ref, b_ref, msg_ref, z_ref):
    h = jnp.maximum(p_ref[0] + p_ref[1] + zp_ref[...], 0.0)
    msg_ref[...] = jnp.dot(h, w1_ref[...], preferred_element_type=_F32,
                           precision=_HI)
    z_ref[...] = jnp.dot(h, w2_ref[...], preferred_element_type=_F32,
                         precision=_HI) + b_ref[...]


def _pool_body(p_ref, zp_ref, seg_ref, wd_ref, bd_ref, out_ref,
               sums_ref, cnt_ref):
    i = pl.program_id(0)

    @pl.when(i == 0)
    def _init():
        sums_ref[...] = jnp.zeros_like(sums_ref)
        cnt_ref[...] = jnp.zeros_like(cnt_ref)

    h = jnp.maximum(p_ref[0] + p_ref[1] + zp_ref[...], 0.0)
    seg = seg_ref[0]
    gids = lax.broadcasted_iota(jnp.int32, (G, RB), 0)
    onehot = (seg == gids).astype(_F32)
    sums_ref[...] += jnp.dot(onehot, h, preferred_element_type=_F32,
                             precision=_HI)
    cnt_ref[...] += jnp.sum(onehot, axis=1, keepdims=True)

    @pl.when(i == NBLK - 1)
    def _finish():
        pooled = sums_ref[...] / jnp.maximum(cnt_ref[...], 1.0)
        logits = jnp.dot(pooled, wd_ref[...], preferred_element_type=_F32,
                         precision=_HI) + bd_ref[...]
        m = jnp.max(logits, axis=1, keepdims=True)
        e = jnp.exp(logits - m)
        out_ref[...] = e / jnp.sum(e, axis=1, keepdims=True)


_dense_first = pl.pallas_call(
    _dense_first_body,
    grid=(NBLK,),
    in_specs=[
        pl.BlockSpec((RB, D), lambda i: (i, 0)),
        pl.BlockSpec((D, H), lambda i: (0, 0)),
        pl.BlockSpec((D, H), lambda i: (0, 0)),
        pl.BlockSpec((1, H), lambda i: (0, 0)),
    ],
    out_specs=[
        pl.BlockSpec((RB, H), lambda i: (i, 0)),
        pl.BlockSpec((RB, H), lambda i: (i, 0)),
    ],
    out_shape=[
        jax.ShapeDtypeStruct((N, H), _F32),
        jax.ShapeDtypeStruct((N, H), _F32),
    ],
)

_dense_mid = pl.pallas_call(
    _dense_mid_body,
    grid=(NBLK,),
    in_specs=[
        pl.BlockSpec((NC, RB, H), lambda i: (0, i, 0)),
        pl.BlockSpec((RB, H), lambda i: (i, 0)),
        pl.BlockSpec((H, H), lambda i: (0, 0)),
        pl.BlockSpec((H, H), lambda i: (0, 0)),
        pl.BlockSpec((1, H), lambda i: (0, 0)),
    ],
    out_specs=[
        pl.BlockSpec((RB, H), lambda i: (i, 0)),
        pl.BlockSpec((RB, H), lambda i: (i, 0)),
    ],
    out_shape=[
        jax.ShapeDtypeStruct((N, H), _F32),
        jax.ShapeDtypeStruct((N, H), _F32),
    ],
)

_pool = pl.pallas_call(
    _pool_body,
    grid=(NBLK,),
    in_specs=[
        pl.BlockSpec((NC, RB, H), lambda i: (0, i, 0)),
        pl.BlockSpec((RB, H), lambda i: (i, 0)),
        pl.BlockSpec((1, 1, RB), lambda i: (i, 0, 0)),
        pl.BlockSpec((H, LC), lambda i: (0, 0)),
        pl.BlockSpec((1, LC), lambda i: (0, 0)),
    ],
    out_specs=pl.BlockSpec((G, LC), lambda i: (0, 0)),
    out_shape=jax.ShapeDtypeStruct((G, LC), _F32),
    scratch_shapes=[
        pltpu.VMEM((G, H), _F32),
        pltpu.VMEM((G, 1), _F32),
    ],
)


MRW = N // NS


def _edge_agg_body(msg_hbm, src_hbm, dst_hbm, w_hbm, zeros_hbm, out_hbm,
                   src_v, dst_v, w_v, rows_v, msg_sh, agg_sh, sem):
    cid = lax.axis_index("c")
    sid = lax.axis_index("s")
    wid = cid * NS + sid

    pltpu.sync_copy(zeros_hbm.at[pl.ds(sid * RPW, RPW)],
                    agg_sh.at[pl.ds(sid * RPW, RPW)])
    pltpu.sync_copy(msg_hbm.at[pl.ds(sid * MRW, MRW)],
                    msg_sh.at[pl.ds(sid * MRW, MRW)])
    pltpu.sync_copy(src_hbm.at[wid], src_v)
    pltpu.sync_copy(dst_hbm.at[wid], dst_v)
    pltpu.sync_copy(w_hbm.at[wid], w_v)
    plsc.subcore_barrier()

    def chunk_body(c, carry):
        pltpu.async_copy(msg_sh.at[src_v.at[c]], rows_v, sem).wait()

        def scale_body(g, acc):
            w16 = w_v[c, pl.ds(g * 16, 16)]
            for j in range(16):
                k = g * 16 + j
                wk = w16[j]
                rows_v[k, pl.ds(0, 16)] = rows_v[k, pl.ds(0, 16)] * wk
                rows_v[k, pl.ds(16, 16)] = rows_v[k, pl.ds(16, 16)] * wk
            return acc
        lax.fori_loop(0, CHUNK // 16, scale_body, 0)

        pltpu.sync_copy(rows_v, agg_sh.at[dst_v.at[c]], add=True)
        return carry

    lax.fori_loop(0, NCHUNK, chunk_body, 0)
    plsc.subcore_barrier()

    pltpu.sync_copy(agg_sh.at[pl.ds(sid * RPW, RPW)],
                    out_hbm.at[cid, pl.ds(sid * RPW, RPW)])


_edge_agg = functools.partial(
    pl.kernel,
    out_type=jax.ShapeDtypeStruct((NC, NPAD, H), _F32),
    mesh=plsc.VectorSubcoreMesh(core_axis_name="c", subcore_axis_name="s"),
    scratch_types=[
        pltpu.VMEM((NCHUNK, CHUNK), jnp.int32),
        pltpu.VMEM((NCHUNK, CHUNK), jnp.int32),
        pltpu.VMEM((NCHUNK, CHUNK), _F32),
        pltpu.VMEM((CHUNK, H), _F32),
        pltpu.VMEM_SHARED((N, H), _F32),
        pltpu.VMEM_SHARED((NPAD, H), _F32),
        pltpu.SemaphoreType.DMA,
    ],
    compiler_params=pltpu.CompilerParams(use_tc_tiling_on_sc=False),
)(_edge_agg_body)


def kernel(x, edge_index, edge_weight, seg_ids, W1a, W2a, ba, W1b, W2b, bb,
           W1c, W2c, bc, Wd, bd):
    pad = EPAD - E
    src = jnp.pad(edge_index[0], (0, pad)).reshape(NW, NCHUNK, CHUNK)
    dst = jnp.pad(edge_index[1], (0, pad)).reshape(NW, NCHUNK, CHUNK)
    w = jnp.pad(edge_weight, (0, pad)).reshape(NW, NCHUNK, CHUNK)
    zeros_nh = jnp.zeros((NPAD, H), _F32)
    seg3 = seg_ids.reshape(NBLK, 1, RB)

    msg, z = _dense_first(x, W1a, W2a, ba.reshape(1, H))
    p = _edge_agg(msg, src, dst, w, zeros_nh)
    msg, z = _dense_mid(p, z, W1b, W2b, bb.reshape(1, H))
    p = _edge_agg(msg, src, dst, w, zeros_nh)
    msg, z = _dense_mid(p, z, W1c, W2c, bc.reshape(1, H))
    p = _edge_agg(msg, src, dst, w, zeros_nh)
    return _pool(p, z, seg3, Wd, bd.reshape(1, LC))


# speedup vs baseline: 1.1678x; 1.1678x over previous
import functools

import jax
import jax.numpy as jnp
from jax import lax
from jax.experimental import pallas as pl
from jax.experimental.pallas import tpu as pltpu
from jax.experimental.pallas import tpu_sc as plsc

N = 10000
D = 128
H = 32
G = 64
LC = 2
E = 320000

NC = 2
NS = 16
NW = NC * NS

CHUNK = 256
NCHUNK = 40
EPW = NCHUNK * CHUNK
EPAD = EPW * NW
NPAD = 10240
RPW = NPAD // NS

RB = 2000
NBLK = N // RB

_F32 = jnp.float32
_HI = lax.Precision.HIGHEST


def _dense_first_body(x_ref, w1_ref, w2_ref, b_ref, msg_ref, z_ref):
    xb = x_ref[...]
    msg_ref[...] = jnp.dot(xb, w1_ref[...], preferred_element_type=_F32,
                           precision=_HI)
    z_ref[...] = jnp.dot(xb, w2_ref[...], preferred_element_type=_F32,
                         precision=_HI) + b_ref[...]


def _dense_mid_body(p_ref, zp_ref, w1_ref, w2_ref, b_ref, msg_ref, z_ref):
    h = jnp.maximum(p_ref[0] + p_ref[1] + zp_ref[...], 0.0)
    msg_ref[...] = jnp.dot(h, w1_ref[...], preferred_element_type=_F32,
                           precision=_HI)
    z_ref[...] = jnp.dot(h, w2_ref[...], preferred_element_type=_F32,
                         precision=_HI) + b_ref[...]


def _pool_body(p_ref, zp_ref, seg_ref, wd_ref, bd_ref, out_ref,
               sums_ref, cnt_ref):
    i = pl.program_id(0)

    @pl.when(i == 0)
    def _init():
        sums_ref[...] = jnp.zeros_like(sums_ref)
        cnt_ref[...] = jnp.zeros_like(cnt_ref)

    h = jnp.maximum(p_ref[0] + p_ref[1] + zp_ref[...], 0.0)
    seg = seg_ref[0]
    gids = lax.broadcasted_iota(jnp.int32, (G, RB), 0)
    onehot = (seg == gids).astype(_F32)
    sums_ref[...] += jnp.dot(onehot, h, preferred_element_type=_F32,
                             precision=_HI)
    cnt_ref[...] += jnp.sum(onehot, axis=1, keepdims=True)

    @pl.when(i == NBLK - 1)
    def _finish():
        pooled = sums_ref[...] / jnp.maximum(cnt_ref[...], 1.0)
        logits = jnp.dot(pooled, wd_ref[...], preferred_element_type=_F32,
                         precision=_HI) + bd_ref[...]
        m = jnp.max(logits, axis=1, keepdims=True)
        e = jnp.exp(logits - m)
        out_ref[...] = e / jnp.sum(e, axis=1, keepdims=True)


_dense_first = pl.pallas_call(
    _dense_first_body,
    grid=(NBLK,),
    in_specs=[
        pl.BlockSpec((RB, D), lambda i: (i, 0)),
        pl.BlockSpec((D, H), lambda i: (0, 0)),
        pl.BlockSpec((D, H), lambda i: (0, 0)),
        pl.BlockSpec((1, H), lambda i: (0, 0)),
    ],
    out_specs=[
        pl.BlockSpec((RB, H), lambda i: (i, 0)),
        pl.BlockSpec((RB, H), lambda i: (i, 0)),
    ],
    out_shape=[
        jax.ShapeDtypeStruct((N, H), _F32),
        jax.ShapeDtypeStruct((N, H), _F32),
    ],
)

_dense_mid = pl.pallas_call(
    _dense_mid_body,
    grid=(NBLK,),
    in_specs=[
        pl.BlockSpec((NC, RB, H), lambda i: (0, i, 0)),
        pl.BlockSpec((RB, H), lambda i: (i, 0)),
        pl.BlockSpec((H, H), lambda i: (0, 0)),
        pl.BlockSpec((H, H), lambda i: (0, 0)),
        pl.BlockSpec((1, H), lambda i: (0, 0)),
    ],
    out_specs=[
        pl.BlockSpec((RB, H), lambda i: (i, 0)),
        pl.BlockSpec((RB, H), lambda i: (i, 0)),
    ],
    out_shape=[
        jax.ShapeDtypeStruct((N, H), _F32),
        jax.ShapeDtypeStruct((N, H), _F32),
    ],
)

_pool = pl.pallas_call(
    _pool_body,
    grid=(NBLK,),
    in_specs=[
        pl.BlockSpec((NC, RB, H), lambda i: (0, i, 0)),
        pl.BlockSpec((RB, H), lambda i: (i, 0)),
        pl.BlockSpec((1, 1, RB), lambda i: (i, 0, 0)),
        pl.BlockSpec((H, LC), lambda i: (0, 0)),
        pl.BlockSpec((1, LC), lambda i: (0, 0)),
    ],
    out_specs=pl.BlockSpec((G, LC), lambda i: (0, 0)),
    out_shape=jax.ShapeDtypeStruct((G, LC), _F32),
    scratch_shapes=[
        pltpu.VMEM((G, H), _F32),
        pltpu.VMEM((G, 1), _F32),
    ],
)


MRW = N // NS


def _edge_agg_body(msg_hbm, src_hbm, dst_hbm, w_hbm, zeros_hbm, out_hbm,
                   src_v, dst_v, w_v, gbuf0, gbuf1, sbuf0, sbuf1,
                   msg_sh, agg_sh, gsem0, gsem1, ssem0, ssem1):
    cid = lax.axis_index("c")
    sid = lax.axis_index("s")
    wid = cid * NS + sid

    pltpu.sync_copy(zeros_hbm.at[pl.ds(sid * RPW, RPW)],
                    agg_sh.at[pl.ds(sid * RPW, RPW)])
    pltpu.sync_copy(msg_hbm.at[pl.ds(sid * MRW, MRW)],
                    msg_sh.at[pl.ds(sid * MRW, MRW)])
    pltpu.sync_copy(src_hbm.at[wid], src_v)
    pltpu.sync_copy(dst_hbm.at[wid], dst_v)
    pltpu.sync_copy(w_hbm.at[wid], w_v)
    plsc.subcore_barrier()

    def scale(c, gb, sb):
        def scale_body(g, acc):
            w16 = w_v[c, pl.ds(g * 16, 16)]
            for j in range(16):
                k = g * 16 + j
                wk = w16[j]
                sb[k, pl.ds(0, 16)] = gb[k, pl.ds(0, 16)] * wk
                sb[k, pl.ds(16, 16)] = gb[k, pl.ds(16, 16)] * wk
            return acc
        lax.fori_loop(0, CHUNK // 16, scale_body, 0)

    def step(c, gb, sb, gsem, ssem, first):
        if not first:
            pltpu.make_async_copy(zeros_hbm.at[pl.ds(0, CHUNK)], sb,
                                  ssem).wait()
        pltpu.make_async_copy(msg_sh.at[src_v.at[c]], gb, gsem).wait()
        scale(c, gb, sb)
        pltpu.async_copy(msg_sh.at[src_v.at[c + 2]], gb, gsem)
        pltpu.async_copy(sb, agg_sh.at[dst_v.at[c]], ssem, add=True)

    pltpu.async_copy(msg_sh.at[src_v.at[0]], gbuf0, gsem0)
    pltpu.async_copy(msg_sh.at[src_v.at[1]], gbuf1, gsem1)
    step(0, gbuf0, sbuf0, gsem0, ssem0, first=True)
    step(1, gbuf1, sbuf1, gsem1, ssem1, first=True)

    def body(c2, acc):
        step(2 * c2, gbuf0, sbuf0, gsem0, ssem0, first=False)
        step(2 * c2 + 1, gbuf1, sbuf1, gsem1, ssem1, first=False)
        return acc
    lax.fori_loop(1, NCHUNK // 2, body, 0)

    pltpu.make_async_copy(zeros_hbm.at[pl.ds(0, CHUNK)], sbuf0, ssem0).wait()
    pltpu.make_async_copy(zeros_hbm.at[pl.ds(0, CHUNK)], sbuf1, ssem1).wait()
    pltpu.make_async_copy(msg_sh.at[src_v.at[NCHUNK]], gbuf0, gsem0).wait()
    pltpu.make_async_copy(msg_sh.at[src_v.at[NCHUNK + 1]], gbuf1,
                          gsem1).wait()
    plsc.subcore_barrier()

    pltpu.sync_copy(agg_sh.at[pl.ds(sid * RPW, RPW)],
                    out_hbm.at[cid, pl.ds(sid * RPW, RPW)])


_edge_agg = functools.partial(
    pl.kernel,
    out_type=jax.ShapeDtypeStruct((NC, NPAD, H), _F32),
    mesh=plsc.VectorSubcoreMesh(core_axis_name="c", subcore_axis_name="s"),
    scratch_types=[
        pltpu.VMEM((NCHUNK + 2, CHUNK), jnp.int32),
        pltpu.VMEM((NCHUNK, CHUNK), jnp.int32),
        pltpu.VMEM((NCHUNK, CHUNK), _F32),
        pltpu.VMEM((CHUNK, H), _F32),
        pltpu.VMEM((CHUNK, H), _F32),
        pltpu.VMEM((CHUNK, H), _F32),
        pltpu.VMEM((CHUNK, H), _F32),
        pltpu.VMEM_SHARED((N, H), _F32),
        pltpu.VMEM_SHARED((NPAD, H), _F32),
        pltpu.SemaphoreType.DMA,
        pltpu.SemaphoreType.DMA,
        pltpu.SemaphoreType.DMA,
        pltpu.SemaphoreType.DMA,
    ],
    compiler_params=pltpu.CompilerParams(use_tc_tiling_on_sc=False),
)(_edge_agg_body)


def kernel(x, edge_index, edge_weight, seg_ids, W1a, W2a, ba, W1b, W2b, bb,
           W1c, W2c, bc, Wd, bd):
    pad = EPAD - E
    src = jnp.pad(edge_index[0], (0, pad)).reshape(NW, NCHUNK, CHUNK)
    src = jnp.pad(src, ((0, 0), (0, 2), (0, 0)))
    dst = jnp.pad(edge_index[1], (0, pad)).reshape(NW, NCHUNK, CHUNK)
    w = jnp.pad(edge_weight, (0, pad)).reshape(NW, NCHUNK, CHUNK)
    zeros_nh = jnp.zeros((NPAD, H), _F32)
    seg3 = seg_ids.reshape(NBLK, 1, RB)

    msg, z = _dense_first(x, W1a, W2a, ba.reshape(1, H))
    p = _edge_agg(msg, src, dst, w, zeros_nh)
    msg, z = _dense_mid(p, z, W1b, W2b, bb.reshape(1, H))
    p = _edge_agg(msg, src, dst, w, zeros_nh)
    msg, z = _dense_mid(p, z, W1c, W2c, bc.reshape(1, H))
    p = _edge_agg(msg, src, dst, w, zeros_nh)
    return _pool(p, z, seg3, Wd, bd.reshape(1, LC))


# speedup vs baseline: 1.4393x; 1.2325x over previous
import functools

import jax
import jax.numpy as jnp
from jax import lax
from jax.experimental import pallas as pl
from jax.experimental.pallas import tpu as pltpu
from jax.experimental.pallas import tpu_sc as plsc

N = 10000
D = 128
H = 32
G = 64
LC = 2
E = 320000

NC = 2
NS = 16
NW = NC * NS

CHUNK = 400
NCHUNK = 25
EPW = NCHUNK * CHUNK
NPAD = 10240
RPW = NPAD // NS
MRW = N // NS

_F32 = jnp.float32


def _dense_first_body(x_ref, w1_ref, w2_ref, b_ref, msg_ref, z_ref):
    xb = x_ref[...]
    msg_ref[...] = jnp.dot(xb, w1_ref[...], preferred_element_type=_F32)
    z_ref[...] = jnp.dot(xb, w2_ref[...], preferred_element_type=_F32) \
        + b_ref[...]


def _dense_mid_body(p_ref, zp_ref, w1_ref, w2_ref, b_ref, msg_ref, z_ref):
    h = jnp.maximum(p_ref[0] + p_ref[1] + zp_ref[...], 0.0)
    msg_ref[...] = jnp.dot(h, w1_ref[...], preferred_element_type=_F32)
    z_ref[...] = jnp.dot(h, w2_ref[...], preferred_element_type=_F32) \
        + b_ref[...]


def _pool_body(p_ref, zp_ref, seg_ref, wd_ref, bd_ref, out_ref):
    h = jnp.maximum(p_ref[0] + p_ref[1] + zp_ref[...], 0.0)
    seg = seg_ref[0]
    gids = lax.broadcasted_iota(jnp.int32, (G, N), 0)
    onehot = (seg == gids).astype(_F32)
    sums = jnp.dot(onehot, h, preferred_element_type=_F32)
    cnt = jnp.sum(onehot, axis=1, keepdims=True)
    pooled = sums / jnp.maximum(cnt, 1.0)
    logits = jnp.dot(pooled, wd_ref[...], preferred_element_type=_F32) \
        + bd_ref[...]
    m = jnp.max(logits, axis=1, keepdims=True)
    e = jnp.exp(logits - m)
    out_ref[...] = e / jnp.sum(e, axis=1, keepdims=True)


_dense_first = pl.pallas_call(
    _dense_first_body,
    out_shape=[
        jax.ShapeDtypeStruct((N, H), _F32),
        jax.ShapeDtypeStruct((N, H), _F32),
    ],
)

_dense_mid = pl.pallas_call(
    _dense_mid_body,
    grid=(1,),
    in_specs=[
        pl.BlockSpec((NC, N, H), lambda i: (0, 0, 0)),
        pl.BlockSpec((N, H), lambda i: (0, 0)),
        pl.BlockSpec((H, H), lambda i: (0, 0)),
        pl.BlockSpec((H, H), lambda i: (0, 0)),
        pl.BlockSpec((1, H), lambda i: (0, 0)),
    ],
    out_specs=[
        pl.BlockSpec((N, H), lambda i: (0, 0)),
        pl.BlockSpec((N, H), lambda i: (0, 0)),
    ],
    out_shape=[
        jax.ShapeDtypeStruct((N, H), _F32),
        jax.ShapeDtypeStruct((N, H), _F32),
    ],
)

_pool = pl.pallas_call(
    _pool_body,
    grid=(1,),
    in_specs=[
        pl.BlockSpec((NC, N, H), lambda i: (0, 0, 0)),
        pl.BlockSpec((N, H), lambda i: (0, 0)),
        pl.BlockSpec((1, 1, N), lambda i: (0, 0, 0)),
        pl.BlockSpec((H, LC), lambda i: (0, 0)),
        pl.BlockSpec((1, LC), lambda i: (0, 0)),
    ],
    out_specs=pl.BlockSpec((G, LC), lambda i: (0, 0)),
    out_shape=jax.ShapeDtypeStruct((G, LC), _F32),
)


def _edge_agg_body(msg_hbm, ei_hbm, w_hbm, zeros_hbm, out_hbm,
                   src_v, dst_v, w_v, gbuf0, gbuf1, sbuf0, sbuf1,
                   msg_sh, agg_sh, gsem0, gsem1, ssem0, ssem1):
    cid = lax.axis_index("c")
    sid = lax.axis_index("s")
    wid = cid * NS + sid

    pltpu.sync_copy(zeros_hbm.at[pl.ds(sid * RPW, RPW)],
                    agg_sh.at[pl.ds(sid * RPW, RPW)])
    pltpu.sync_copy(msg_hbm.at[pl.ds(sid * MRW, MRW)],
                    msg_sh.at[pl.ds(sid * MRW, MRW)])
    pltpu.sync_copy(ei_hbm.at[0, pl.ds(wid * EPW, EPW)], src_v)
    pltpu.sync_copy(ei_hbm.at[1, pl.ds(wid * EPW, EPW)], dst_v)
    pltpu.sync_copy(w_hbm.at[pl.ds(wid * EPW, EPW)], w_v)
    plsc.subcore_barrier()

    def scale(c, gb, sb):
        def scale_body(g, acc):
            w16 = w_v[pl.ds(c * CHUNK + g * 16, 16)]
            for j in range(16):
                k = g * 16 + j
                wk = w16[j]
                sb[k, pl.ds(0, 16)] = gb[k, pl.ds(0, 16)] * wk
                sb[k, pl.ds(16, 16)] = gb[k, pl.ds(16, 16)] * wk
            return acc
        lax.fori_loop(0, CHUNK // 16, scale_body, 0)

    def gidx(c):
        cc = jnp.minimum(c, NCHUNK - 1)
        return src_v.at[pl.ds(cc * CHUNK, CHUNK)]

    def step(c, gb, sb, gsem, ssem, first):
        if not first:
            pltpu.make_async_copy(zeros_hbm.at[pl.ds(0, CHUNK)], sb,
                                  ssem).wait()
        pltpu.make_async_copy(msg_sh.at[gidx(c)], gb, gsem).wait()
        scale(c, gb, sb)
        pltpu.async_copy(msg_sh.at[gidx(c + 2)], gb, gsem)
        pltpu.async_copy(sb, agg_sh.at[dst_v.at[pl.ds(c * CHUNK, CHUNK)]],
                         ssem, add=True)

    pltpu.async_copy(msg_sh.at[gidx(0)], gbuf0, gsem0)
    pltpu.async_copy(msg_sh.at[gidx(1)], gbuf1, gsem1)
    step(0, gbuf0, sbuf0, gsem0, ssem0, first=True)
    step(1, gbuf1, sbuf1, gsem1, ssem1, first=True)

    def body(c2, acc):
        step(2 * c2, gbuf0, sbuf0, gsem0, ssem0, first=False)
        step(2 * c2 + 1, gbuf1, sbuf1, gsem1, ssem1, first=False)
        return acc
    lax.fori_loop(1, NCHUNK // 2, body, 0)

    step(NCHUNK - 1, gbuf0, sbuf0, gsem0, ssem0, first=False)

    pltpu.make_async_copy(zeros_hbm.at[pl.ds(0, CHUNK)], sbuf0, ssem0).wait()
    pltpu.make_async_copy(zeros_hbm.at[pl.ds(0, CHUNK)], sbuf1, ssem1).wait()
    pltpu.make_async_copy(msg_sh.at[gidx(NCHUNK)], gbuf0, gsem0).wait()
    pltpu.make_async_copy(msg_sh.at[gidx(NCHUNK + 1)], gbuf1, gsem1).wait()
    plsc.subcore_barrier()

    pltpu.sync_copy(agg_sh.at[pl.ds(sid * RPW, RPW)],
                    out_hbm.at[cid, pl.ds(sid * RPW, RPW)])


_edge_agg = functools.partial(
    pl.kernel,
    out_type=jax.ShapeDtypeStruct((NC, NPAD, H), _F32),
    mesh=plsc.VectorSubcoreMesh(core_axis_name="c", subcore_axis_name="s"),
    scratch_types=[
        pltpu.VMEM((EPW,), jnp.int32),
        pltpu.VMEM((EPW,), jnp.int32),
        pltpu.VMEM((EPW,), _F32),
        pltpu.VMEM((CHUNK, H), _F32),
        pltpu.VMEM((CHUNK, H), _F32),
        pltpu.VMEM((CHUNK, H), _F32),
        pltpu.VMEM((CHUNK, H), _F32),
        pltpu.VMEM_SHARED((N, H), _F32),
        pltpu.VMEM_SHARED((NPAD, H), _F32),
        pltpu.SemaphoreType.DMA,
        pltpu.SemaphoreType.DMA,
        pltpu.SemaphoreType.DMA,
        pltpu.SemaphoreType.DMA,
    ],
    compiler_params=pltpu.CompilerParams(use_tc_tiling_on_sc=False),
)(_edge_agg_body)


def kernel(x, edge_index, edge_weight, seg_ids, W1a, W2a, ba, W1b, W2b, bb,
           W1c, W2c, bc, Wd, bd):
    zeros_nh = jnp.zeros((NPAD, H), _F32)
    seg3 = seg_ids.reshape(1, 1, N)

    msg, z = _dense_first(x, W1a, W2a, ba.reshape(1, H))
    p = _edge_agg(msg, edge_index, edge_weight, zeros_nh)
    msg, z = _dense_mid(p, z, W1b, W2b, bb.reshape(1, H))
    p = _edge_agg(msg, edge_index, edge_weight, zeros_nh)
    msg, z = _dense_mid(p, z, W1c, W2c, bc.reshape(1, H))
    p = _edge_agg(msg, edge_index, edge_weight, zeros_nh)
    return _pool(p, z, seg3, Wd, bd.reshape(1, LC))


# speedup vs baseline: 1.8403x; 1.2786x over previous
import functools

import jax
import jax.numpy as jnp
from jax import lax
from jax.experimental import pallas as pl
from jax.experimental.pallas import tpu as pltpu
from jax.experimental.pallas import tpu_sc as plsc

N = 10000
D = 128
H = 32
G = 64
LC = 2
E = 320000

NC = 2
NS = 16
NW = NC * NS

CHUNK = 400
NCHUNK = 25
EPW = NCHUNK * CHUNK
NPAD = 10240
RPW = NPAD // NS
MRW = NPAD // NS

_F32 = jnp.float32


NP4 = NPAD // 4
N4 = N // 4


def _dense_first_body(x4_ref, w1_ref, w2_ref, b_ref, msg_ref, z_ref):
    xb = x4_ref[...]
    msg = jnp.dot(xb, w1_ref[...], preferred_element_type=_F32)
    z = jnp.dot(xb, w2_ref[...], preferred_element_type=_F32) + b_ref[...]
    msg_ref[0:N4] = msg
    msg_ref[N4:NP4] = jnp.zeros((NP4 - N4, 4 * H), _F32)
    z_ref[0:N4] = z
    z_ref[N4:NP4] = jnp.zeros((NP4 - N4, 4 * H), _F32)


def _dense_mid_body(p_ref, zp_ref, w1_ref, w2_ref, b_ref, msg_ref, z_ref):
    hp = jnp.maximum(p_ref[0] + p_ref[1] + zp_ref[...], 0.0)
    msg_ref[...] = jnp.dot(hp, w1_ref[...], preferred_element_type=_F32)
    z_ref[...] = jnp.dot(hp, w2_ref[...], preferred_element_type=_F32) \
        + b_ref[...]


def _pool_body(p_ref, zp_ref, seg_ref, wd_ref, bd_ref, out_ref):
    hp = jnp.maximum(p_ref[0] + p_ref[1] + zp_ref[...], 0.0)
    gids = lax.broadcasted_iota(jnp.int32, (G, NP4), 0)
    sums = jnp.zeros((G, H), _F32)
    cnt = jnp.zeros((G, 1), _F32)
    for r in range(4):
        seg_r = seg_ref[r:r + 1, :]
        onehot = (seg_r == gids).astype(_F32)
        full = jnp.dot(onehot, hp, preferred_element_type=_F32)
        sums = sums + lax.slice(full, (0, r * H), (G, (r + 1) * H))
        cnt = cnt + jnp.sum(onehot, axis=1, keepdims=True)
    pooled = sums / jnp.maximum(cnt, 1.0)
    logits = jnp.dot(pooled, wd_ref[...], preferred_element_type=_F32) \
        + bd_ref[...]
    m = jnp.max(logits, axis=1, keepdims=True)
    e = jnp.exp(logits - m)
    out_ref[...] = e / jnp.sum(e, axis=1, keepdims=True)


_dense_first = pl.pallas_call(
    _dense_first_body,
    out_shape=[
        jax.ShapeDtypeStruct((NP4, 4 * H), _F32),
        jax.ShapeDtypeStruct((NP4, 4 * H), _F32),
    ],
)

_dense_mid = pl.pallas_call(
    _dense_mid_body,
    out_shape=[
        jax.ShapeDtypeStruct((NP4, 4 * H), _F32),
        jax.ShapeDtypeStruct((NP4, 4 * H), _F32),
    ],
)

_pool = pl.pallas_call(
    _pool_body,
    out_shape=jax.ShapeDtypeStruct((G, LC), _F32),
)


def _edge_agg_body(msg_hbm, ei_hbm, w_hbm, zeros_hbm, out_hbm,
                   src_v, dst_v, w_v, gbuf0, gbuf1, sbuf0, sbuf1,
                   msg_sh, agg_sh, gsem0, gsem1, ssem0, ssem1):
    cid = lax.axis_index("c")
    sid = lax.axis_index("s")
    wid = cid * NS + sid

    pltpu.sync_copy(zeros_hbm.at[pl.ds(sid * RPW, RPW)],
                    agg_sh.at[pl.ds(sid * RPW, RPW)])
    pltpu.sync_copy(msg_hbm.at[pl.ds(sid * MRW, MRW)],
                    msg_sh.at[pl.ds(sid * MRW, MRW)])
    pltpu.sync_copy(ei_hbm.at[0, pl.ds(wid * EPW, EPW)], src_v)
    pltpu.sync_copy(ei_hbm.at[1, pl.ds(wid * EPW, EPW)], dst_v)
    pltpu.sync_copy(w_hbm.at[pl.ds(wid * EPW, EPW)], w_v)
    plsc.subcore_barrier()

    def scale(c, gb, sb):
        def scale_body(g, acc):
            w16 = w_v[pl.ds(c * CHUNK + g * 16, 16)]
            for j in range(16):
                k = g * 16 + j
                wk = w16[j]
                sb[k, pl.ds(0, 16)] = gb[k, pl.ds(0, 16)] * wk
                sb[k, pl.ds(16, 16)] = gb[k, pl.ds(16, 16)] * wk
            return acc
        lax.fori_loop(0, CHUNK // 16, scale_body, 0)

    def gidx(c):
        cc = jnp.minimum(c, NCHUNK - 1)
        return src_v.at[pl.ds(cc * CHUNK, CHUNK)]

    def step(c, gb, sb, gsem, ssem, first):
        if not first:
            pltpu.make_async_copy(zeros_hbm.at[pl.ds(0, CHUNK)], sb,
                                  ssem).wait()
        pltpu.make_async_copy(msg_sh.at[gidx(c)], gb, gsem).wait()
        scale(c, gb, sb)
        pltpu.async_copy(msg_sh.at[gidx(c + 2)], gb, gsem)
        pltpu.async_copy(sb, agg_sh.at[dst_v.at[pl.ds(c * CHUNK, CHUNK)]],
                         ssem, add=True)

    pltpu.async_copy(msg_sh.at[gidx(0)], gbuf0, gsem0)
    pltpu.async_copy(msg_sh.at[gidx(1)], gbuf1, gsem1)
    step(0, gbuf0, sbuf0, gsem0, ssem0, first=True)
    step(1, gbuf1, sbuf1, gsem1, ssem1, first=True)

    def body(c2, acc):
        step(2 * c2, gbuf0, sbuf0, gsem0, ssem0, first=False)
        step(2 * c2 + 1, gbuf1, sbuf1, gsem1, ssem1, first=False)
        return acc
    lax.fori_loop(1, NCHUNK // 2, body, 0)

    step(NCHUNK - 1, gbuf0, sbuf0, gsem0, ssem0, first=False)

    pltpu.make_async_copy(zeros_hbm.at[pl.ds(0, CHUNK)], sbuf0, ssem0).wait()
    pltpu.make_async_copy(zeros_hbm.at[pl.ds(0, CHUNK)], sbuf1, ssem1).wait()
    pltpu.make_async_copy(msg_sh.at[gidx(NCHUNK)], gbuf0, gsem0).wait()
    pltpu.make_async_copy(msg_sh.at[gidx(NCHUNK + 1)], gbuf1, gsem1).wait()
    plsc.subcore_barrier()

    pltpu.sync_copy(agg_sh.at[pl.ds(sid * RPW, RPW)],
                    out_hbm.at[cid, pl.ds(sid * RPW, RPW)])


_edge_agg = functools.partial(
    pl.kernel,
    out_type=jax.ShapeDtypeStruct((NC, NPAD, H), _F32),
    mesh=plsc.VectorSubcoreMesh(core_axis_name="c", subcore_axis_name="s"),
    scratch_types=[
        pltpu.VMEM((EPW,), jnp.int32),
        pltpu.VMEM((EPW,), jnp.int32),
        pltpu.VMEM((EPW,), _F32),
        pltpu.VMEM((CHUNK, H), _F32),
        pltpu.VMEM((CHUNK, H), _F32),
        pltpu.VMEM((CHUNK, H), _F32),
        pltpu.VMEM((CHUNK, H), _F32),
        pltpu.VMEM_SHARED((NPAD, H), _F32),
        pltpu.VMEM_SHARED((NPAD, H), _F32),
        pltpu.SemaphoreType.DMA,
        pltpu.SemaphoreType.DMA,
        pltpu.SemaphoreType.DMA,
        pltpu.SemaphoreType.DMA,
    ],
    compiler_params=pltpu.CompilerParams(use_tc_tiling_on_sc=False),
)(_edge_agg_body)


def kernel(x, edge_index, edge_weight, seg_ids, W1a, W2a, ba, W1b, W2b, bb,
           W1c, W2c, bc, Wd, bd):
    zeros_nh = jnp.zeros((NPAD, H), _F32)
    eye4 = jnp.eye(4, dtype=_F32)

    x4 = x.reshape(N // 4, 4 * D)
    w1a_bd = jnp.kron(eye4, W1a)
    w2a_bd = jnp.kron(eye4, W2a)
    w1b_bd = jnp.kron(eye4, W1b)
    w2b_bd = jnp.kron(eye4, W2b)
    w1c_bd = jnp.kron(eye4, W1c)
    w2c_bd = jnp.kron(eye4, W2c)
    ba4 = jnp.tile(ba, 4).reshape(1, 4 * H)
    bb4 = jnp.tile(bb, 4).reshape(1, 4 * H)
    bc4 = jnp.tile(bc, 4).reshape(1, 4 * H)
    segp = jnp.pad(seg_ids.reshape(N // 4, 4).T,
                   ((0, 4), (0, (NPAD - N) // 4)), constant_values=-1)

    def sc_view(msgp):
        return msgp.reshape(NPAD, H)

    def tc_view(p):
        return p.reshape(NC, NPAD // 4, 4 * H)

    msgp, zp = _dense_first(x4, w1a_bd, w2a_bd, ba4)
    p = _edge_agg(sc_view(msgp), edge_index, edge_weight, zeros_nh)
    msgp, zp = _dense_mid(tc_view(p), zp, w1b_bd, w2b_bd, bb4)
    p = _edge_agg(sc_view(msgp), edge_index, edge_weight, zeros_nh)
    msgp, zp = _dense_mid(tc_view(p), zp, w1c_bd, w2c_bd, bc4)
    p = _edge_agg(sc_view(msgp), edge_index, edge_weight, zeros_nh)
    return _pool(tc_view(p), zp, segp, Wd, bd.reshape(1, LC))


# speedup vs baseline: 1.9839x; 1.0780x over previous
import functools

import jax
import jax.numpy as jnp
from jax import lax
from jax.experimental import pallas as pl
from jax.experimental.pallas import tpu as pltpu
from jax.experimental.pallas import tpu_sc as plsc

N = 10000
D = 128
H = 32
G = 64
LC = 2
E = 320000

NC = 2
NS = 16
NW = NC * NS

CHUNK = 200
NCHUNK = 50
EPW = NCHUNK * CHUNK
NPAD = 10240
RPW = NPAD // NS
MRW = NPAD // NS

_F32 = jnp.float32


NP4 = NPAD // 4
N4 = N // 4


def _dense_first_body(x4_ref, w1_ref, w2_ref, b_ref, msg_ref, z_ref):
    xb = x4_ref[...]
    msg = jnp.dot(xb, w1_ref[...], preferred_element_type=_F32)
    z = jnp.dot(xb, w2_ref[...], preferred_element_type=_F32) + b_ref[...]
    msg_ref[0:N4] = msg
    msg_ref[N4:NP4] = jnp.zeros((NP4 - N4, 4 * H), _F32)
    z_ref[0:N4] = z
    z_ref[N4:NP4] = jnp.zeros((NP4 - N4, 4 * H), _F32)


def _dense_mid_body(p_ref, zp_ref, w1_ref, w2_ref, b_ref, msg_ref, z_ref):
    hp = jnp.maximum(p_ref[0] + p_ref[1] + zp_ref[...], 0.0)
    msg_ref[...] = jnp.dot(hp, w1_ref[...], preferred_element_type=_F32)
    z_ref[...] = jnp.dot(hp, w2_ref[...], preferred_element_type=_F32) \
        + b_ref[...]


def _pool_body(p_ref, zp_ref, seg_ref, wd_ref, bd_ref, out_ref):
    hp = jnp.maximum(p_ref[0] + p_ref[1] + zp_ref[...], 0.0)
    gids = lax.broadcasted_iota(jnp.int32, (G, NP4), 0)
    sums = jnp.zeros((G, H), _F32)
    cnt = jnp.zeros((G, 1), _F32)
    for r in range(4):
        seg_r = seg_ref[r:r + 1, :]
        onehot = (seg_r == gids).astype(_F32)
        full = jnp.dot(onehot, hp, preferred_element_type=_F32)
        sums = sums + lax.slice(full, (0, r * H), (G, (r + 1) * H))
        cnt = cnt + jnp.sum(onehot, axis=1, keepdims=True)
    pooled = sums / jnp.maximum(cnt, 1.0)
    logits = jnp.dot(pooled, wd_ref[...], preferred_element_type=_F32) \
        + bd_ref[...]
    m = jnp.max(logits, axis=1, keepdims=True)
    e = jnp.exp(logits - m)
    out_ref[...] = e / jnp.sum(e, axis=1, keepdims=True)


_dense_first = pl.pallas_call(
    _dense_first_body,
    out_shape=[
        jax.ShapeDtypeStruct((NP4, 4 * H), _F32),
        jax.ShapeDtypeStruct((NP4, 4 * H), _F32),
    ],
)

_dense_mid = pl.pallas_call(
    _dense_mid_body,
    out_shape=[
        jax.ShapeDtypeStruct((NP4, 4 * H), _F32),
        jax.ShapeDtypeStruct((NP4, 4 * H), _F32),
    ],
)

_pool = pl.pallas_call(
    _pool_body,
    out_shape=jax.ShapeDtypeStruct((G, LC), _F32),
)


def _edge_agg_body(msg_hbm, ei_hbm, w_hbm, zeros_hbm, out_hbm,
                   src_v, dst_v, w_v, gb0, gb1, gb2, sb0, sb1, sb2,
                   msg_sh, agg_sh, stg_sem,
                   gsem0, gsem1, gsem2, ssem0, ssem1, ssem2):
    cid = lax.axis_index("c")
    sid = lax.axis_index("s")
    wid = cid * NS + sid
    gbufs = (gb0, gb1, gb2)
    sbufs = (sb0, sb1, sb2)
    gsems = (gsem0, gsem1, gsem2)
    ssems = (ssem0, ssem1, ssem2)

    c1 = pltpu.async_copy(zeros_hbm.at[pl.ds(sid * RPW, RPW)],
                          agg_sh.at[pl.ds(sid * RPW, RPW)], stg_sem)
    c2 = pltpu.async_copy(msg_hbm.at[pl.ds(sid * MRW, MRW)],
                          msg_sh.at[pl.ds(sid * MRW, MRW)], stg_sem)
    c3 = pltpu.async_copy(ei_hbm.at[0, pl.ds(wid * EPW, EPW)], src_v, stg_sem)
    c4 = pltpu.async_copy(ei_hbm.at[1, pl.ds(wid * EPW, EPW)], dst_v, stg_sem)
    c5 = pltpu.async_copy(w_hbm.at[pl.ds(wid * EPW, EPW)], w_v, stg_sem)
    c1.wait()
    c2.wait()
    c3.wait()
    c4.wait()
    c5.wait()
    plsc.subcore_barrier()

    def scale(c, gb, sb):
        def scale_body(g, acc):
            w16 = w_v[pl.ds(c * CHUNK + g * 16, 16)]
            for j in range(16):
                k = g * 16 + j
                wk = w16[j]
                sb[k, pl.ds(0, 16)] = gb[k, pl.ds(0, 16)] * wk
                sb[k, pl.ds(16, 16)] = gb[k, pl.ds(16, 16)] * wk
            return acc
        lax.fori_loop(0, CHUNK // 16, scale_body, 0)

    def gidx(c):
        cc = jnp.minimum(c, NCHUNK - 1)
        return src_v.at[pl.ds(cc * CHUNK, CHUNK)]

    def step(c, r, first):
        gb, sb = gbufs[r], sbufs[r]
        gsem, ssem = gsems[r], ssems[r]
        if not first:
            pltpu.make_async_copy(zeros_hbm.at[pl.ds(0, CHUNK)], sb,
                                  ssem).wait()
        pltpu.make_async_copy(msg_sh.at[gidx(c)], gb, gsem).wait()
        scale(c, gb, sb)
        pltpu.async_copy(msg_sh.at[gidx(c + 3)], gb, gsem)
        pltpu.async_copy(sb, agg_sh.at[dst_v.at[pl.ds(c * CHUNK, CHUNK)]],
                         ssem, add=True)

    pltpu.async_copy(msg_sh.at[gidx(0)], gb0, gsem0)
    pltpu.async_copy(msg_sh.at[gidx(1)], gb1, gsem1)
    pltpu.async_copy(msg_sh.at[gidx(2)], gb2, gsem2)
    step(0, 0, first=True)
    step(1, 1, first=True)
    step(2, 2, first=True)

    def body(c3b, acc):
        c = 3 * c3b
        step(c, 0, first=False)
        step(c + 1, 1, first=False)
        step(c + 2, 2, first=False)
        return acc
    lax.fori_loop(1, NCHUNK // 3, body, 0)

    step(NCHUNK - 2, 0, first=False)
    step(NCHUNK - 1, 1, first=False)

    pltpu.make_async_copy(zeros_hbm.at[pl.ds(0, CHUNK)], sb0, ssem0).wait()
    pltpu.make_async_copy(zeros_hbm.at[pl.ds(0, CHUNK)], sb1, ssem1).wait()
    pltpu.make_async_copy(zeros_hbm.at[pl.ds(0, CHUNK)], sb2, ssem2).wait()
    pltpu.make_async_copy(msg_sh.at[gidx(NCHUNK)], gb0, gsem0).wait()
    pltpu.make_async_copy(msg_sh.at[gidx(NCHUNK)], gb1, gsem1).wait()
    pltpu.make_async_copy(msg_sh.at[gidx(NCHUNK)], gb2, gsem2).wait()
    plsc.subcore_barrier()

    pltpu.sync_copy(agg_sh.at[pl.ds(sid * RPW, RPW)],
                    out_hbm.at[cid, pl.ds(sid * RPW, RPW)])


_edge_agg = functools.partial(
    pl.kernel,
    out_type=jax.ShapeDtypeStruct((NC, NPAD, H), _F32),
    mesh=plsc.VectorSubcoreMesh(core_axis_name="c", subcore_axis_name="s"),
    scratch_types=[
        pltpu.VMEM((EPW,), jnp.int32),
        pltpu.VMEM((EPW,), jnp.int32),
        pltpu.VMEM((EPW,), _F32),
        pltpu.VMEM((CHUNK, H), _F32),
        pltpu.VMEM((CHUNK, H), _F32),
        pltpu.VMEM((CHUNK, H), _F32),
        pltpu.VMEM((CHUNK, H), _F32),
        pltpu.VMEM((CHUNK, H), _F32),
        pltpu.VMEM((CHUNK, H), _F32),
        pltpu.VMEM_SHARED((NPAD, H), _F32),
        pltpu.VMEM_SHARED((NPAD, H), _F32),
        pltpu.SemaphoreType.DMA,
        pltpu.SemaphoreType.DMA,
        pltpu.SemaphoreType.DMA,
        pltpu.SemaphoreType.DMA,
        pltpu.SemaphoreType.DMA,
        pltpu.SemaphoreType.DMA,
        pltpu.SemaphoreType.DMA,
    ],
    compiler_params=pltpu.CompilerParams(use_tc_tiling_on_sc=False),
)(_edge_agg_body)


def kernel(x, edge_index, edge_weight, seg_ids, W1a, W2a, ba, W1b, W2b, bb,
           W1c, W2c, bc, Wd, bd):
    zeros_nh = jnp.zeros((NPAD, H), _F32)
    eye4 = jnp.eye(4, dtype=_F32)

    x4 = x.reshape(N // 4, 4 * D)
    w1a_bd = jnp.kron(eye4, W1a)
    w2a_bd = jnp.kron(eye4, W2a)
    w1b_bd = jnp.kron(eye4, W1b)
    w2b_bd = jnp.kron(eye4, W2b)
    w1c_bd = jnp.kron(eye4, W1c)
    w2c_bd = jnp.kron(eye4, W2c)
    ba4 = jnp.tile(ba, 4).reshape(1, 4 * H)
    bb4 = jnp.tile(bb, 4).reshape(1, 4 * H)
    bc4 = jnp.tile(bc, 4).reshape(1, 4 * H)
    segp = jnp.pad(seg_ids.reshape(N // 4, 4).T,
                   ((0, 4), (0, (NPAD - N) // 4)), constant_values=-1)

    def sc_view(msgp):
        return msgp.reshape(NPAD, H)

    def tc_view(p):
        return p.reshape(NC, NPAD // 4, 4 * H)

    msgp, zp = _dense_first(x4, w1a_bd, w2a_bd, ba4)
    p = _edge_agg(sc_view(msgp), edge_index, edge_weight, zeros_nh)
    msgp, zp = _dense_mid(tc_view(p), zp, w1b_bd, w2b_bd, bb4)
    p = _edge_agg(sc_view(msgp), edge_index, edge_weight, zeros_nh)
    msgp, zp = _dense_mid(tc_view(p), zp, w1c_bd, w2c_bd, bc4)
    p = _edge_agg(sc_view(msgp), edge_index, edge_weight, zeros_nh)
    return _pool(tc_view(p), zp, segp, Wd, bd.reshape(1, LC))
